# baseline (speedup 1.0000x reference)
import functools

import jax
import jax.numpy as jnp
from jax.experimental import pallas as pl
from jax.experimental.pallas import tpu as pltpu

B = 4096
D = 768
C1 = 40
C2 = 40
NC = C1 + C2
MARGIN = 1.0
EPS = 1e-6
BB = 512
NB = B // BB


def _rank_weights(pT, wsig):
    CK = 8
    pj = pT[None, :, :]
    ij = jax.lax.broadcasted_iota(jnp.int32, (1, C1, 1), 1)
    rank = jnp.zeros(pT.shape, jnp.float32)
    for c in range(0, C1, CK):
        pk = pT[c:c + CK, None, :]
        ik = c + jax.lax.broadcasted_iota(jnp.int32, (CK, 1, 1), 0)
        mask = jnp.where(ik < ij, 1.0, 0.0)
        tie = jnp.where(pk == pj, mask, 0.0)
        ahead = jnp.where(pk > pj, 1.0, tie)
        rank = rank + jnp.sum(ahead, axis=0)
    ri = rank.astype(jnp.int32)
    bit = [(ri & (1 << k)) != 0 for k in range(4)]
    t = [jnp.where(bit[0], wsig[2 * i + 1], wsig[2 * i]) for i in range(20)]
    t = [jnp.where(bit[1], t[2 * i + 1], t[2 * i]) for i in range(10)]
    t = [jnp.where(bit[2], t[2 * i + 1], t[2 * i]) for i in range(5)]
    u = [jnp.where(bit[3], t[1], t[0]), jnp.where(bit[3], t[3], t[2]), t[4]]
    return jnp.where(ri >= 32, u[2], jnp.where(ri >= 16, u[1], u[0]))


def _loss_kernel(stu_ref, p1_ref, p2_ref, cw_ref, labT_ref, w_ref, out_ref):
    ustar = stu_ref[...] + EPS
    cw = cw_ref[...]
    p1T = p1_ref[...]
    p2T = p2_ref[...]

    dots = jax.lax.dot_general(
        cw, ustar, (((1,), (1,)), ((), ())),
        preferred_element_type=jnp.float32)
    u2 = jax.lax.dot_general(
        jnp.ones((1, D), jnp.float32), ustar * ustar, (((1,), (1,)), ((), ())),
        preferred_element_type=jnp.float32)
    v2 = jnp.sum(cw * cw, axis=1, keepdims=True)
    dn = jnp.sqrt(jnp.maximum(u2 - 2.0 * dots + v2, 0.0))

    lab = labT_ref[...]
    row = jax.lax.broadcasted_iota(jnp.int32, (NC, 1), 0)
    dp = jnp.sum(jnp.where(row == lab, dn, 0.0), axis=0, keepdims=True)

    hinge = jnp.maximum(dp - dn + MARGIN, 0.0)

    wsig = [jnp.float32(0.0)] + [w_ref[0, r] for r in range(1, C1)]
    wsel1 = _rank_weights(p1T, wsig)
    wsel2 = _rank_weights(p2T, wsig)

    block_sum = jnp.sum(wsel1 * hinge[:C1, :] +
                        wsel2 * hinge[C1:, :]).reshape(1, 1)

    @pl.when(pl.program_id(0) == 0)
    def _init():
        out_ref[...] = jnp.zeros((1, 1), jnp.float32)

    out_ref[...] += block_sum

    @pl.when(pl.program_id(0) == NB - 1)
    def _finish():
        out_ref[...] = out_ref[...] * (1.0 / B)


@functools.partial(jax.jit, static_argnames=("interpret",))
def kernel(stu_emb, t1_prob, t2_prob, classifier_weight, labels, weights_param,
           interpret=False):
    out = pl.pallas_call(
        _loss_kernel,
        grid=(NB,),
        in_specs=[
            pl.BlockSpec((BB, D), lambda i: (i, 0)),
            pl.BlockSpec((C1, BB), lambda i: (0, i)),
            pl.BlockSpec((C2, BB), lambda i: (0, i)),
            pl.BlockSpec((NC, D), lambda i: (0, 0)),
            pl.BlockSpec((1, BB), lambda i: (0, i)),
            pl.BlockSpec(memory_space=pltpu.SMEM),
        ],
        out_specs=pl.BlockSpec((1, 1), lambda i: (0, 0)),
        out_shape=jax.ShapeDtypeStruct((1, 1), jnp.float32),
        interpret=interpret,
    )(
        stu_emb,
        t1_prob.T,
        t2_prob.T,
        classifier_weight,
        labels.astype(jnp.int32).reshape(1, B),
        jax.nn.sigmoid(weights_param).reshape(1, C1),
    )
    return out.reshape(())


# speedup vs baseline: 1.0509x; 1.0509x over previous
import functools

import jax
import jax.numpy as jnp
from jax.experimental import pallas as pl
from jax.experimental.pallas import tpu as pltpu

B = 4096
D = 768
C1 = 40
C2 = 40
NC = C1 + C2
MARGIN = 1.0
EPS = 1e-6
BB = 512
NB = B // BB


def _rank_weights(pT, wsig):
    CK = 8
    pj = pT[None, :, :]
    ij = jax.lax.broadcasted_iota(jnp.int32, (1, C1, 1), 1)
    rank = jnp.zeros(pT.shape, jnp.float32)
    for c in range(0, C1, CK):
        pk = pT[c:c + CK, None, :]
        ik = c + jax.lax.broadcasted_iota(jnp.int32, (CK, 1, 1), 0)
        ge = jnp.where(pk >= pj, 1.0, 0.0)
        gt = jnp.where(pk > pj, 1.0, 0.0)
        ahead = jnp.where(ik < ij, ge, gt)
        rank = rank + jnp.sum(ahead, axis=0)
    ri = rank.astype(jnp.int32)
    bit = [(ri & (1 << k)) != 0 for k in range(4)]
    t = [jnp.where(bit[0], wsig[2 * i + 1], wsig[2 * i]) for i in range(20)]
    t = [jnp.where(bit[1], t[2 * i + 1], t[2 * i]) for i in range(10)]
    t = [jnp.where(bit[2], t[2 * i + 1], t[2 * i]) for i in range(5)]
    u = [jnp.where(bit[3], t[1], t[0]), jnp.where(bit[3], t[3], t[2]), t[4]]
    return jnp.where(ri >= 32, u[2], jnp.where(ri >= 16, u[1], u[0]))


def _loss_kernel(stu_ref, p1_ref, p2_ref, cw_ref, labT_ref, w_ref, out_ref):
    ustar = stu_ref[...] + EPS
    cw = cw_ref[...]
    p1T = p1_ref[...]
    p2T = p2_ref[...]

    dots = jax.lax.dot_general(
        cw, ustar, (((1,), (1,)), ((), ())),
        preferred_element_type=jnp.float32)
    u2 = jax.lax.dot_general(
        jnp.ones((1, D), jnp.float32), ustar * ustar, (((1,), (1,)), ((), ())),
        preferred_element_type=jnp.float32)
    v2 = jnp.sum(cw * cw, axis=1, keepdims=True)
    dn = jnp.sqrt(jnp.maximum(u2 - 2.0 * dots + v2, 0.0))

    lab = labT_ref[...]
    row = jax.lax.broadcasted_iota(jnp.int32, (NC, 1), 0)
    dp = jnp.sum(jnp.where(row == lab, dn, 0.0), axis=0, keepdims=True)

    hinge = jnp.maximum(dp - dn + MARGIN, 0.0)

    wsig = [jnp.float32(0.0)] + [w_ref[0, r] for r in range(1, C1)]
    wsel1 = _rank_weights(p1T, wsig)
    wsel2 = _rank_weights(p2T, wsig)

    block_sum = jnp.sum(wsel1 * hinge[:C1, :] +
                        wsel2 * hinge[C1:, :]).reshape(1, 1)

    @pl.when(pl.program_id(0) == 0)
    def _init():
        out_ref[...] = jnp.zeros((1, 1), jnp.float32)

    out_ref[...] += block_sum

    @pl.when(pl.program_id(0) == NB - 1)
    def _finish():
        out_ref[...] = out_ref[...] * (1.0 / B)


@functools.partial(jax.jit, static_argnames=("interpret",))
def kernel(stu_emb, t1_prob, t2_prob, classifier_weight, labels, weights_param,
           interpret=False):
    out = pl.pallas_call(
        _loss_kernel,
        grid=(NB,),
        in_specs=[
            pl.BlockSpec((BB, D), lambda i: (i, 0)),
            pl.BlockSpec((C1, BB), lambda i: (0, i)),
            pl.BlockSpec((C2, BB), lambda i: (0, i)),
            pl.BlockSpec((NC, D), lambda i: (0, 0)),
            pl.BlockSpec((1, BB), lambda i: (0, i)),
            pl.BlockSpec(memory_space=pltpu.SMEM),
        ],
        out_specs=pl.BlockSpec((1, 1), lambda i: (0, 0)),
        out_shape=jax.ShapeDtypeStruct((1, 1), jnp.float32),
        interpret=interpret,
    )(
        stu_emb,
        t1_prob.T,
        t2_prob.T,
        classifier_weight,
        labels.astype(jnp.int32).reshape(1, B),
        jax.nn.sigmoid(weights_param).reshape(1, C1),
    )
    return out.reshape(())
